# Initial kernel scaffold; baseline (speedup 1.0000x reference)
import jax
import jax.numpy as jnp
from jax.experimental import pallas as pl


def kernel(feature, edge_index, edge_type, W_in, b_in, Wg1, bg1, Wg2, bg2, W_out, b_out):
    raise NotImplementedError("write your pallas kernel here")


# speedup vs baseline: 12.5270x; 12.5270x over previous
import functools

import jax
import jax.numpy as jnp
from jax import lax
from jax.experimental import pallas as pl
from jax.experimental.pallas import tpu as pltpu
from jax.experimental.pallas import tpu_sc as plsc

N = 10000
D = 128
E = 320000
OUT = 3

NC = 2
NS = 16
NW = NC * NS
EPT = E // NW
CHUNK = 80
NCHUNK = EPT // CHUNK
NPAD = 10240
RPT = NPAD // NS
DEGW = 128

_mesh = plsc.VectorSubcoreMesh(
    core_axis_name="c", subcore_axis_name="s", num_cores=NC, num_subcores=NS
)


@functools.partial(
    pl.kernel,
    out_type=jax.ShapeDtypeStruct((NC, NPAD, DEGW), jnp.float32),
    mesh=_mesh,
    scratch_types=[
        pltpu.VMEM((CHUNK,), jnp.int32),
        pltpu.VMEM((CHUNK, DEGW), jnp.float32),
        pltpu.VMEM_SHARED((NPAD, DEGW), jnp.float32),
    ],
)
def _deg_kernel(dst_hbm, ones_hbm, zeros_hbm, deg_out, idx_v, ones_v, acc_sh):
    c = lax.axis_index("c")
    s = lax.axis_index("s")
    wid = s * NC + c
    pltpu.sync_copy(ones_hbm, ones_v)
    pltpu.sync_copy(zeros_hbm, acc_sh.at[pl.ds(s * RPT, RPT)])
    plsc.subcore_barrier()

    def body(j, carry):
        base = wid * EPT + j * CHUNK
        pltpu.sync_copy(dst_hbm.at[pl.ds(base, CHUNK)], idx_v)
        pltpu.sync_copy(ones_v, acc_sh.at[idx_v], add=True)
        return carry

    lax.fori_loop(0, NCHUNK, body, 0)
    plsc.subcore_barrier()
    pltpu.sync_copy(
        acc_sh.at[pl.ds(s * RPT, RPT)], deg_out.at[c, pl.ds(s * RPT, RPT)]
    )


@functools.partial(
    pl.kernel,
    out_type=jax.ShapeDtypeStruct((NC, NPAD, D), jnp.float32),
    mesh=_mesh,
    scratch_types=[
        pltpu.VMEM((CHUNK,), jnp.int32),
        pltpu.VMEM((CHUNK,), jnp.int32),
        pltpu.VMEM((CHUNK, D), jnp.float32),
        pltpu.VMEM_SHARED((NPAD, D), jnp.float32),
        pltpu.SemaphoreType.DMA,
    ],
)
def _prop_kernel(y_hbm, src_hbm, dst_hbm, zeros_hbm, z_out, srcv, dstv, rows_v,
                 acc_sh, sem):
    c = lax.axis_index("c")
    s = lax.axis_index("s")
    wid = s * NC + c
    pltpu.sync_copy(zeros_hbm, acc_sh.at[pl.ds(s * RPT, RPT)])
    plsc.subcore_barrier()

    def body(j, carry):
        base = wid * EPT + j * CHUNK
        pltpu.sync_copy(src_hbm.at[pl.ds(base, CHUNK)], srcv)
        pltpu.sync_copy(dst_hbm.at[pl.ds(base, CHUNK)], dstv)
        pltpu.async_copy(y_hbm.at[srcv], rows_v, sem).wait()
        pltpu.sync_copy(rows_v, acc_sh.at[dstv], add=True)
        return carry

    lax.fori_loop(0, NCHUNK, body, 0)
    plsc.subcore_barrier()
    pltpu.sync_copy(
        acc_sh.at[pl.ds(s * RPT, RPT)], z_out.at[c, pl.ds(s * RPT, RPT)]
    )


def _tc_in_body(feat_ref, w_ref, b_ref, degp_ref, y0_ref, dinv_ref):
    degp = degp_ref[...]
    deg = 1.0 + degp[0, :N, 0] + degp[1, :N, 0]
    dinv = lax.rsqrt(deg)[:, None]
    h = jnp.dot(feat_ref[...], w_ref[...].T, preferred_element_type=jnp.float32)
    h = h + b_ref[...][None, :]
    h = jnp.where(h >= 0, h, 0.01 * h)
    y0_ref[...] = dinv * h
    dinv_ref[...] = dinv


def _tc_mid_body(zp_ref, y_ref, dinv_ref, w_ref, b_ref, out_ref):
    dinv = dinv_ref[...]
    zp = zp_ref[...]
    prop = dinv * (zp[0, :N] + zp[1, :N] + y_ref[...])
    x = jnp.dot(prop, w_ref[...].T, preferred_element_type=jnp.float32)
    out_ref[...] = dinv * (x + b_ref[...][None, :])


def _tc_final_body(zp_ref, y_ref, dinv_ref, wg_ref, bg_ref, wo_ref, bo_ref,
                   out_ref):
    dinv = dinv_ref[...]
    zp = zp_ref[...]
    prop = dinv * (zp[0, :N] + zp[1, :N] + y_ref[...])
    x = jnp.dot(prop, wg_ref[...].T, preferred_element_type=jnp.float32)
    x = x + bg_ref[...][None, :]
    o = jnp.dot(x, wo_ref[...].T, preferred_element_type=jnp.float32)
    out_ref[...] = o + bo_ref[...][None, :]


def kernel(feature, edge_index, edge_type, W_in, b_in, Wg1, bg1, Wg2, bg2,
           W_out, b_out):
    del edge_type
    src = edge_index[0].astype(jnp.int32)
    dst = edge_index[1].astype(jnp.int32)
    ones_w = jnp.ones((CHUNK, DEGW), jnp.float32)
    zeros_d = jnp.zeros((RPT, D), jnp.float32)

    degp = _deg_kernel(dst, ones_w, zeros_d)

    y0, dinv = pl.pallas_call(
        _tc_in_body,
        out_shape=[
            jax.ShapeDtypeStruct((N, D), jnp.float32),
            jax.ShapeDtypeStruct((N, 1), jnp.float32),
        ],
    )(feature, W_in, b_in, degp)

    zp1 = _prop_kernel(y0, src, dst, zeros_d)

    y1 = pl.pallas_call(
        _tc_mid_body,
        out_shape=jax.ShapeDtypeStruct((N, D), jnp.float32),
    )(zp1, y0, dinv, Wg1, bg1)

    zp2 = _prop_kernel(y1, src, dst, zeros_d)

    out = pl.pallas_call(
        _tc_final_body,
        out_shape=jax.ShapeDtypeStruct((N, OUT), jnp.float32),
    )(zp2, y1, dinv, Wg2, bg2, W_out, b_out)
    return out


# speedup vs baseline: 18.0643x; 1.4420x over previous
import functools

import jax
import jax.numpy as jnp
from jax import lax
from jax.experimental import pallas as pl
from jax.experimental.pallas import tpu as pltpu
from jax.experimental.pallas import tpu_sc as plsc

N = 10000
D = 128
E = 320000
OUT = 3

NC = 2
NS = 16
NW = NC * NS
EPT = E // NW
CHUNK = 80
NBUF = 4
NGRP = EPT // (CHUNK * NBUF)
NPAD = 10240
RPT = NPAD // NS

_mesh = plsc.VectorSubcoreMesh(
    core_axis_name="c", subcore_axis_name="s", num_cores=NC, num_subcores=NS
)


@functools.partial(
    pl.kernel,
    out_type=jax.ShapeDtypeStruct((NC, NPAD, D), jnp.float32),
    mesh=_mesh,
    scratch_types=[
        pltpu.VMEM((CHUNK,), jnp.int32),
        pltpu.VMEM((CHUNK, D), jnp.float32),
        pltpu.VMEM_SHARED((NPAD, D), jnp.float32),
    ],
)
def _deg_kernel(dst_hbm, ones_hbm, zeros_hbm, deg_out, dstv, ones_v, acc_sh):
    c = lax.axis_index("c")
    s = lax.axis_index("s")
    wid = s * NC + c
    pltpu.sync_copy(ones_hbm, ones_v)
    pltpu.sync_copy(zeros_hbm, acc_sh.at[pl.ds(s * RPT, RPT)])
    plsc.subcore_barrier()

    def body(j, carry):
        base = wid * EPT + j * CHUNK
        pltpu.sync_copy(dst_hbm.at[pl.ds(base, CHUNK)], dstv)
        pltpu.sync_copy(ones_v, acc_sh.at[dstv], add=True)
        return carry

    lax.fori_loop(0, EPT // CHUNK, body, 0)
    plsc.subcore_barrier()
    pltpu.sync_copy(
        acc_sh.at[pl.ds(s * RPT, RPT)], deg_out.at[c, pl.ds(s * RPT, RPT)]
    )


@functools.partial(
    pl.kernel,
    out_type=jax.ShapeDtypeStruct((NC, NPAD, D), jnp.float32),
    mesh=_mesh,
    scratch_types=[
        [pltpu.VMEM((CHUNK,), jnp.int32)] * NBUF,
        [pltpu.VMEM((CHUNK,), jnp.int32)] * NBUF,
        [pltpu.VMEM((CHUNK, D), jnp.float32)] * NBUF,
        pltpu.VMEM_SHARED((NPAD, D), jnp.float32),
        [pltpu.SemaphoreType.DMA] * NBUF,
    ],
)
def _prop_kernel(y_hbm, src_hbm, dst_hbm, zeros_hbm, z_out, srcs, dsts,
                 rows, acc_sh, sems):
    c = lax.axis_index("c")
    s = lax.axis_index("s")
    wid = s * NC + c
    pltpu.sync_copy(zeros_hbm, acc_sh.at[pl.ds(s * RPT, RPT)])
    plsc.subcore_barrier()

    def body(j, carry):
        descs = []
        for b in range(NBUF):
            base = wid * EPT + (j * NBUF + b) * CHUNK
            pltpu.sync_copy(src_hbm.at[pl.ds(base, CHUNK)], srcs[b])
            pltpu.sync_copy(dst_hbm.at[pl.ds(base, CHUNK)], dsts[b])
            descs.append(pltpu.async_copy(y_hbm.at[srcs[b]], rows[b], sems[b]))
        for b in range(NBUF):
            descs[b].wait()
            pltpu.sync_copy(rows[b], acc_sh.at[dsts[b]], add=True)
        return carry

    lax.fori_loop(0, NGRP, body, 0)
    for t in range(NGRP * NBUF, EPT // CHUNK):
        base = wid * EPT + t * CHUNK
        pltpu.sync_copy(src_hbm.at[pl.ds(base, CHUNK)], srcs[0])
        pltpu.sync_copy(dst_hbm.at[pl.ds(base, CHUNK)], dsts[0])
        pltpu.async_copy(y_hbm.at[srcs[0]], rows[0], sems[0]).wait()
        pltpu.sync_copy(rows[0], acc_sh.at[dsts[0]], add=True)
    plsc.subcore_barrier()
    pltpu.sync_copy(
        acc_sh.at[pl.ds(s * RPT, RPT)], z_out.at[c, pl.ds(s * RPT, RPT)]
    )


def _tc_in_body(feat_ref, w_ref, b_ref, degp_ref, y0_ref, dinv_ref):
    degp = degp_ref[...]
    deg = 1.0 + degp[0, :N, 0] + degp[1, :N, 0]
    dinv = lax.rsqrt(deg)[:, None]
    h = jnp.dot(feat_ref[...], w_ref[...].T, preferred_element_type=jnp.float32)
    h = h + b_ref[...][None, :]
    h = jnp.where(h >= 0, h, 0.01 * h)
    y0_ref[...] = dinv * h
    dinv_ref[...] = dinv


def _tc_mid_body(zp_ref, y_ref, dinv_ref, w_ref, b_ref, out_ref):
    dinv = dinv_ref[...]
    zp = zp_ref[...]
    prop = dinv * (zp[0, :N] + zp[1, :N] + y_ref[...])
    x = jnp.dot(prop, w_ref[...].T, preferred_element_type=jnp.float32)
    out_ref[...] = dinv * (x + b_ref[...][None, :])


def _tc_final_body(zp_ref, y_ref, dinv_ref, wg_ref, bg_ref, wo_ref, bo_ref,
                   out_ref):
    dinv = dinv_ref[...]
    zp = zp_ref[...]
    prop = dinv * (zp[0, :N] + zp[1, :N] + y_ref[...])
    x = jnp.dot(prop, wg_ref[...].T, preferred_element_type=jnp.float32)
    x = x + bg_ref[...][None, :]
    o = jnp.dot(x, wo_ref[...].T, preferred_element_type=jnp.float32)
    out_ref[...] = o + bo_ref[...][None, :]


def kernel(feature, edge_index, edge_type, W_in, b_in, Wg1, bg1, Wg2, bg2,
           W_out, b_out):
    del edge_type
    src = edge_index[0].astype(jnp.int32)
    dst = edge_index[1].astype(jnp.int32)
    zeros_d = jnp.zeros((RPT, D), jnp.float32)
    ones_w = jnp.ones((CHUNK, D), jnp.float32)

    degp = _deg_kernel(dst, ones_w, zeros_d)

    y0, dinv = pl.pallas_call(
        _tc_in_body,
        out_shape=[
            jax.ShapeDtypeStruct((N, D), jnp.float32),
            jax.ShapeDtypeStruct((N, 1), jnp.float32),
        ],
    )(feature, W_in, b_in, degp)

    zp1 = _prop_kernel(y0, src, dst, zeros_d)

    y1 = pl.pallas_call(
        _tc_mid_body,
        out_shape=jax.ShapeDtypeStruct((N, D), jnp.float32),
    )(zp1, y0, dinv, Wg1, bg1)

    zp2 = _prop_kernel(y1, src, dst, zeros_d)

    out = pl.pallas_call(
        _tc_final_body,
        out_shape=jax.ShapeDtypeStruct((N, OUT), jnp.float32),
    )(zp2, y1, dinv, Wg2, bg2, W_out, b_out)
    return out


# speedup vs baseline: 18.5590x; 1.0274x over previous
import functools

import jax
import jax.numpy as jnp
from jax import lax
from jax.experimental import pallas as pl
from jax.experimental.pallas import tpu as pltpu
from jax.experimental.pallas import tpu_sc as plsc

N = 10000
D = 128
E = 320000
OUT = 3

NC = 2
NS = 16
NW = NC * NS
EPT = E // NW
CHUNK = 80
NBUF = 4
NGRP = EPT // (CHUNK * NBUF)
NPAD = 10240
RPT = NPAD // NS

_mesh = plsc.VectorSubcoreMesh(
    core_axis_name="c", subcore_axis_name="s", num_cores=NC, num_subcores=NS
)


@functools.partial(
    pl.kernel,
    out_type=jax.ShapeDtypeStruct((NC, NPAD, D), jnp.float32),
    mesh=_mesh,
    scratch_types=[
        [pltpu.VMEM((CHUNK,), jnp.int32)] * 5,
        pltpu.VMEM((CHUNK, D), jnp.float32),
        pltpu.VMEM_SHARED((NPAD, D), jnp.float32),
        [pltpu.SemaphoreType.DMA] * 5,
    ],
)
def _deg_kernel(dst_hbm, ones_hbm, zeros_hbm, deg_out, dsts, ones_v, acc_sh,
                sems):
    c = lax.axis_index("c")
    s = lax.axis_index("s")
    wid = s * NC + c
    pltpu.sync_copy(ones_hbm, ones_v)
    pltpu.sync_copy(zeros_hbm, acc_sh.at[pl.ds(s * RPT, RPT)])
    plsc.subcore_barrier()

    def body(j, carry):
        for b in range(5):
            base = wid * EPT + (j * 5 + b) * CHUNK
            pltpu.sync_copy(dst_hbm.at[pl.ds(base, CHUNK)], dsts[b])
        descs = [
            pltpu.async_copy(ones_v, acc_sh.at[dsts[b]], sems[b], add=True)
            for b in range(5)
        ]
        for d in descs:
            d.wait()
        return carry

    lax.fori_loop(0, EPT // (CHUNK * 5), body, 0)
    plsc.subcore_barrier()
    pltpu.sync_copy(
        acc_sh.at[pl.ds(s * RPT, RPT)], deg_out.at[c, pl.ds(s * RPT, RPT)]
    )


@functools.partial(
    pl.kernel,
    out_type=jax.ShapeDtypeStruct((NC, NPAD, D), jnp.float32),
    mesh=_mesh,
    scratch_types=[
        [pltpu.VMEM((CHUNK,), jnp.int32)] * NBUF,
        [pltpu.VMEM((CHUNK,), jnp.int32)] * NBUF,
        [pltpu.VMEM((CHUNK, D), jnp.float32)] * NBUF,
        pltpu.VMEM_SHARED((NPAD, D), jnp.float32),
        [pltpu.SemaphoreType.DMA] * NBUF,
        [pltpu.SemaphoreType.DMA] * NBUF,
    ],
)
def _prop_kernel(y_hbm, src_hbm, dst_hbm, zeros_hbm, z_out, srcs, dsts,
                 rows, acc_sh, sems, ssems):
    c = lax.axis_index("c")
    s = lax.axis_index("s")
    wid = s * NC + c
    pltpu.sync_copy(zeros_hbm, acc_sh.at[pl.ds(s * RPT, RPT)])
    plsc.subcore_barrier()

    def body(j, carry):
        descs = []
        for b in range(NBUF):
            base = wid * EPT + (j * NBUF + b) * CHUNK
            pltpu.sync_copy(src_hbm.at[pl.ds(base, CHUNK)], srcs[b])
            pltpu.sync_copy(dst_hbm.at[pl.ds(base, CHUNK)], dsts[b])
            descs.append(pltpu.async_copy(y_hbm.at[srcs[b]], rows[b], sems[b]))
        sdescs = []
        for b in range(NBUF):
            descs[b].wait()
            sdescs.append(
                pltpu.async_copy(rows[b], acc_sh.at[dsts[b]], ssems[b],
                                 add=True)
            )
        for d in sdescs:
            d.wait()
        return carry

    lax.fori_loop(0, NGRP, body, 0)
    for t in range(NGRP * NBUF, EPT // CHUNK):
        base = wid * EPT + t * CHUNK
        pltpu.sync_copy(src_hbm.at[pl.ds(base, CHUNK)], srcs[0])
        pltpu.sync_copy(dst_hbm.at[pl.ds(base, CHUNK)], dsts[0])
        pltpu.async_copy(y_hbm.at[srcs[0]], rows[0], sems[0]).wait()
        pltpu.sync_copy(rows[0], acc_sh.at[dsts[0]], add=True)
    plsc.subcore_barrier()
    pltpu.sync_copy(
        acc_sh.at[pl.ds(s * RPT, RPT)], z_out.at[c, pl.ds(s * RPT, RPT)]
    )


def _tc_in_body(feat_ref, w_ref, b_ref, degp_ref, y0_ref, dinv_ref):
    degp = degp_ref[...]
    deg = 1.0 + degp[0, :N, 0] + degp[1, :N, 0]
    dinv = lax.rsqrt(deg)[:, None]
    h = jnp.dot(feat_ref[...], w_ref[...].T, preferred_element_type=jnp.float32)
    h = h + b_ref[...][None, :]
    h = jnp.where(h >= 0, h, 0.01 * h)
    y0_ref[...] = dinv * h
    dinv_ref[...] = dinv


def _tc_mid_body(zp_ref, y_ref, dinv_ref, w_ref, b_ref, out_ref):
    dinv = dinv_ref[...]
    zp = zp_ref[...]
    prop = dinv * (zp[0, :N] + zp[1, :N] + y_ref[...])
    x = jnp.dot(prop, w_ref[...].T, preferred_element_type=jnp.float32)
    out_ref[...] = dinv * (x + b_ref[...][None, :])


def _tc_final_body(zp_ref, y_ref, dinv_ref, wg_ref, bg_ref, wo_ref, bo_ref,
                   out_ref):
    dinv = dinv_ref[...]
    zp = zp_ref[...]
    prop = dinv * (zp[0, :N] + zp[1, :N] + y_ref[...])
    x = jnp.dot(prop, wg_ref[...].T, preferred_element_type=jnp.float32)
    x = x + bg_ref[...][None, :]
    o = jnp.dot(x, wo_ref[...].T, preferred_element_type=jnp.float32)
    out_ref[...] = o + bo_ref[...][None, :]


def kernel(feature, edge_index, edge_type, W_in, b_in, Wg1, bg1, Wg2, bg2,
           W_out, b_out):
    del edge_type
    src = edge_index[0].astype(jnp.int32)
    dst = edge_index[1].astype(jnp.int32)
    zeros_d = jnp.zeros((RPT, D), jnp.float32)
    ones_w = jnp.ones((CHUNK, D), jnp.float32)

    degp = _deg_kernel(dst, ones_w, zeros_d)

    y0, dinv = pl.pallas_call(
        _tc_in_body,
        out_shape=[
            jax.ShapeDtypeStruct((N, D), jnp.float32),
            jax.ShapeDtypeStruct((N, 1), jnp.float32),
        ],
    )(feature, W_in, b_in, degp)

    zp1 = _prop_kernel(y0, src, dst, zeros_d)

    y1 = pl.pallas_call(
        _tc_mid_body,
        out_shape=jax.ShapeDtypeStruct((N, D), jnp.float32),
    )(zp1, y0, dinv, Wg1, bg1)

    zp2 = _prop_kernel(y1, src, dst, zeros_d)

    out = pl.pallas_call(
        _tc_final_body,
        out_shape=jax.ShapeDtypeStruct((N, OUT), jnp.float32),
    )(zp2, y1, dinv, Wg2, bg2, W_out, b_out)
    return out


# speedup vs baseline: 20.1904x; 1.0879x over previous
import functools

import jax
import jax.numpy as jnp
from jax import lax
from jax.experimental import pallas as pl
from jax.experimental.pallas import tpu as pltpu
from jax.experimental.pallas import tpu_sc as plsc

N = 10000
D = 128
E = 320000
OUT = 3

NC = 2
NS = 16
NW = NC * NS
EPT = E // NW
CHUNK = 80
NBUF = 2
NGRP = EPT // (CHUNK * NBUF)
NPAD = 10240
RPT = NPAD // NS

_mesh = plsc.VectorSubcoreMesh(
    core_axis_name="c", subcore_axis_name="s", num_cores=NC, num_subcores=NS
)


@functools.partial(
    pl.kernel,
    out_type=jax.ShapeDtypeStruct((NC, NPAD, D), jnp.float32),
    mesh=_mesh,
    scratch_types=[
        [pltpu.VMEM((CHUNK,), jnp.int32)] * 5,
        pltpu.VMEM((CHUNK, D), jnp.float32),
        pltpu.VMEM_SHARED((NPAD, D), jnp.float32),
        [pltpu.SemaphoreType.DMA] * 5,
    ],
)
def _deg_kernel(dst_hbm, ones_hbm, zeros_hbm, deg_out, dsts, ones_v, acc_sh,
                sems):
    c = lax.axis_index("c")
    s = lax.axis_index("s")
    wid = s * NC + c
    pltpu.sync_copy(ones_hbm, ones_v)
    pltpu.sync_copy(zeros_hbm, acc_sh.at[pl.ds(s * RPT, RPT)])
    plsc.subcore_barrier()

    def body(j, carry):
        for b in range(5):
            base = wid * EPT + (j * 5 + b) * CHUNK
            pltpu.sync_copy(dst_hbm.at[pl.ds(base, CHUNK)], dsts[b])
        descs = [
            pltpu.async_copy(ones_v, acc_sh.at[dsts[b]], sems[b], add=True)
            for b in range(5)
        ]
        for d in descs:
            d.wait()
        return carry

    lax.fori_loop(0, EPT // (CHUNK * 5), body, 0)
    plsc.subcore_barrier()
    pltpu.sync_copy(
        acc_sh.at[pl.ds(s * RPT, RPT)], deg_out.at[c, pl.ds(s * RPT, RPT)]
    )


@functools.partial(
    pl.kernel,
    out_type=jax.ShapeDtypeStruct((NC, NPAD, D), jnp.float32),
    mesh=_mesh,
    scratch_types=[
        pltpu.VMEM((EPT,), jnp.int32),
        pltpu.VMEM((EPT,), jnp.int32),
        [pltpu.VMEM((CHUNK, D), jnp.float32)] * NBUF,
        pltpu.VMEM_SHARED((NPAD, D), jnp.float32),
        [pltpu.SemaphoreType.DMA] * NBUF,
        [pltpu.SemaphoreType.DMA] * NBUF,
    ],
)
def _prop_kernel(y_hbm, src_hbm, dst_hbm, zeros_hbm, z_out, srcv, dstv,
                 rows, acc_sh, sems, ssems):
    c = lax.axis_index("c")
    s = lax.axis_index("s")
    wid = s * NC + c
    pltpu.sync_copy(src_hbm.at[pl.ds(wid * EPT, EPT)], srcv)
    pltpu.sync_copy(dst_hbm.at[pl.ds(wid * EPT, EPT)], dstv)
    pltpu.sync_copy(zeros_hbm, acc_sh.at[pl.ds(s * RPT, RPT)])
    plsc.subcore_barrier()

    def body(j, carry):
        descs = []
        for b in range(NBUF):
            off = (j * NBUF + b) * CHUNK
            descs.append(
                pltpu.async_copy(
                    y_hbm.at[srcv.at[pl.ds(off, CHUNK)]], rows[b], sems[b]
                )
            )
        sdescs = []
        for b in range(NBUF):
            off = (j * NBUF + b) * CHUNK
            descs[b].wait()
            sdescs.append(
                pltpu.async_copy(
                    rows[b], acc_sh.at[dstv.at[pl.ds(off, CHUNK)]], ssems[b],
                    add=True,
                )
            )
        for d in sdescs:
            d.wait()
        return carry

    lax.fori_loop(0, NGRP, body, 0)
    for t in range(NGRP * NBUF, EPT // CHUNK):
        off = t * CHUNK
        pltpu.async_copy(
            y_hbm.at[srcv.at[pl.ds(off, CHUNK)]], rows[0], sems[0]
        ).wait()
        pltpu.sync_copy(rows[0], acc_sh.at[dstv.at[pl.ds(off, CHUNK)]],
                        add=True)
    plsc.subcore_barrier()
    pltpu.sync_copy(
        acc_sh.at[pl.ds(s * RPT, RPT)], z_out.at[c, pl.ds(s * RPT, RPT)]
    )


def _tc_in_body(feat_ref, w_ref, b_ref, degp_ref, y0_ref, dinv_ref):
    degp = degp_ref[...]
    deg = 1.0 + degp[0, :N, 0] + degp[1, :N, 0]
    dinv = lax.rsqrt(deg)[:, None]
    h = jnp.dot(feat_ref[...], w_ref[...].T, preferred_element_type=jnp.float32)
    h = h + b_ref[...][None, :]
    h = jnp.where(h >= 0, h, 0.01 * h)
    y0_ref[...] = dinv * h
    dinv_ref[...] = dinv


def _tc_mid_body(zp_ref, y_ref, dinv_ref, w_ref, b_ref, out_ref):
    dinv = dinv_ref[...]
    zp = zp_ref[...]
    prop = dinv * (zp[0, :N] + zp[1, :N] + y_ref[...])
    x = jnp.dot(prop, w_ref[...].T, preferred_element_type=jnp.float32)
    out_ref[...] = dinv * (x + b_ref[...][None, :])


def _tc_final_body(zp_ref, y_ref, dinv_ref, wg_ref, bg_ref, wo_ref, bo_ref,
                   out_ref):
    dinv = dinv_ref[...]
    zp = zp_ref[...]
    prop = dinv * (zp[0, :N] + zp[1, :N] + y_ref[...])
    x = jnp.dot(prop, wg_ref[...].T, preferred_element_type=jnp.float32)
    x = x + bg_ref[...][None, :]
    o = jnp.dot(x, wo_ref[...].T, preferred_element_type=jnp.float32)
    out_ref[...] = o + bo_ref[...][None, :]


def kernel(feature, edge_index, edge_type, W_in, b_in, Wg1, bg1, Wg2, bg2,
           W_out, b_out):
    del edge_type
    src = edge_index[0].astype(jnp.int32)
    dst = edge_index[1].astype(jnp.int32)
    zeros_d = jnp.zeros((RPT, D), jnp.float32)
    ones_w = jnp.ones((CHUNK, D), jnp.float32)

    degp = _deg_kernel(dst, ones_w, zeros_d)

    y0, dinv = pl.pallas_call(
        _tc_in_body,
        out_shape=[
            jax.ShapeDtypeStruct((N, D), jnp.float32),
            jax.ShapeDtypeStruct((N, 1), jnp.float32),
        ],
    )(feature, W_in, b_in, degp)

    zp1 = _prop_kernel(y0, src, dst, zeros_d)

    y1 = pl.pallas_call(
        _tc_mid_body,
        out_shape=jax.ShapeDtypeStruct((N, D), jnp.float32),
    )(zp1, y0, dinv, Wg1, bg1)

    zp2 = _prop_kernel(y1, src, dst, zeros_d)

    out = pl.pallas_call(
        _tc_final_body,
        out_shape=jax.ShapeDtypeStruct((N, OUT), jnp.float32),
    )(zp2, y1, dinv, Wg2, bg2, W_out, b_out)
    return out


# speedup vs baseline: 22.2616x; 1.1026x over previous
import functools

import jax
import jax.numpy as jnp
from jax import lax
from jax.experimental import pallas as pl
from jax.experimental.pallas import tpu as pltpu
from jax.experimental.pallas import tpu_sc as plsc

N = 10000
D = 128
E = 320000
OUT = 3

NC = 2
NS = 16
NW = NC * NS
EPT = E // NW
CHUNK = 80
NBUF = 2
NGRP = EPT // (CHUNK * NBUF)
NPAD = 10240
RPT = NPAD // NS

_mesh = plsc.VectorSubcoreMesh(
    core_axis_name="c", subcore_axis_name="s", num_cores=NC, num_subcores=NS
)


@functools.partial(
    pl.kernel,
    out_type=jax.ShapeDtypeStruct((NC, NPAD, D), jnp.float32),
    mesh=_mesh,
    scratch_types=[
        pltpu.VMEM((EPT,), jnp.int32),
        pltpu.VMEM((CHUNK, D), jnp.float32),
        pltpu.VMEM_SHARED((NPAD, D), jnp.float32),
        [pltpu.SemaphoreType.DMA] * 5,
    ],
)
def _deg_kernel(dst_hbm, ones_hbm, zeros_hbm, deg_out, dstv, ones_v, acc_sh,
                sems):
    c = lax.axis_index("c")
    s = lax.axis_index("s")
    wid = s * NC + c
    pltpu.sync_copy(ones_hbm, ones_v)
    pltpu.sync_copy(dst_hbm.at[pl.ds(wid * EPT, EPT)], dstv)
    pltpu.sync_copy(zeros_hbm, acc_sh.at[pl.ds(s * RPT, RPT)])
    plsc.subcore_barrier()

    def body(j, carry):
        descs = [
            pltpu.async_copy(
                ones_v,
                acc_sh.at[dstv.at[pl.ds((j * 5 + b) * CHUNK, CHUNK)]],
                sems[b], add=True,
            )
            for b in range(5)
        ]
        for d in descs:
            d.wait()
        return carry

    lax.fori_loop(0, EPT // (CHUNK * 5), body, 0)
    plsc.subcore_barrier()
    pltpu.sync_copy(
        acc_sh.at[pl.ds(s * RPT, RPT)], deg_out.at[c, pl.ds(s * RPT, RPT)]
    )


@functools.partial(
    pl.kernel,
    out_type=jax.ShapeDtypeStruct((NC, NPAD, D), jnp.float32),
    mesh=_mesh,
    scratch_types=[
        pltpu.VMEM((EPT,), jnp.int32),
        pltpu.VMEM((EPT,), jnp.int32),
        [pltpu.VMEM((CHUNK, D), jnp.float32)] * NBUF,
        pltpu.VMEM_SHARED((NPAD, D), jnp.float32),
        [pltpu.SemaphoreType.DMA] * NBUF,
        [pltpu.SemaphoreType.DMA] * NBUF,
    ],
)
def _prop_kernel(y_hbm, src_hbm, dst_hbm, zeros_hbm, z_out, srcv, dstv,
                 rows, acc_sh, sems, ssems):
    c = lax.axis_index("c")
    s = lax.axis_index("s")
    wid = s * NC + c
    pltpu.sync_copy(src_hbm.at[pl.ds(wid * EPT, EPT)], srcv)
    pltpu.sync_copy(dst_hbm.at[pl.ds(wid * EPT, EPT)], dstv)
    pltpu.sync_copy(zeros_hbm, acc_sh.at[pl.ds(s * RPT, RPT)])
    plsc.subcore_barrier()

    def body(j, carry):
        descs = []
        for b in range(NBUF):
            off = (j * NBUF + b) * CHUNK
            descs.append(
                pltpu.async_copy(
                    y_hbm.at[srcv.at[pl.ds(off, CHUNK)]], rows[b], sems[b]
                )
            )
        sdescs = []
        for b in range(NBUF):
            off = (j * NBUF + b) * CHUNK
            descs[b].wait()
            sdescs.append(
                pltpu.async_copy(
                    rows[b], acc_sh.at[dstv.at[pl.ds(off, CHUNK)]], ssems[b],
                    add=True,
                )
            )
        for d in sdescs:
            d.wait()
        return carry

    lax.fori_loop(0, NGRP, body, 0)
    for t in range(NGRP * NBUF, EPT // CHUNK):
        off = t * CHUNK
        pltpu.async_copy(
            y_hbm.at[srcv.at[pl.ds(off, CHUNK)]], rows[0], sems[0]
        ).wait()
        pltpu.sync_copy(rows[0], acc_sh.at[dstv.at[pl.ds(off, CHUNK)]],
                        add=True)
    plsc.subcore_barrier()
    pltpu.sync_copy(
        acc_sh.at[pl.ds(s * RPT, RPT)], z_out.at[c, pl.ds(s * RPT, RPT)]
    )


def _tc_in_body(feat_ref, w_ref, b_ref, degp_ref, y0_ref, dinv_ref):
    degp = degp_ref[...]
    deg = 1.0 + degp[0, :N, 0] + degp[1, :N, 0]
    dinv = lax.rsqrt(deg)[:, None]
    h = jnp.dot(feat_ref[...], w_ref[...].T, preferred_element_type=jnp.float32)
    h = h + b_ref[...][None, :]
    h = jnp.where(h >= 0, h, 0.01 * h)
    y0_ref[...] = dinv * h
    dinv_ref[...] = dinv


def _tc_mid_body(zp_ref, y_ref, dinv_ref, w_ref, b_ref, out_ref):
    dinv = dinv_ref[...]
    zp = zp_ref[...]
    prop = dinv * (zp[0, :N] + zp[1, :N] + y_ref[...])
    x = jnp.dot(prop, w_ref[...].T, preferred_element_type=jnp.float32)
    out_ref[...] = dinv * (x + b_ref[...][None, :])


def _tc_final_body(zp_ref, y_ref, dinv_ref, wg_ref, bg_ref, wo_ref, bo_ref,
                   out_ref):
    dinv = dinv_ref[...]
    zp = zp_ref[...]
    prop = dinv * (zp[0, :N] + zp[1, :N] + y_ref[...])
    x = jnp.dot(prop, wg_ref[...].T, preferred_element_type=jnp.float32)
    x = x + bg_ref[...][None, :]
    o = jnp.dot(x, wo_ref[...].T, preferred_element_type=jnp.float32)
    out_ref[...] = o + bo_ref[...][None, :]


def kernel(feature, edge_index, edge_type, W_in, b_in, Wg1, bg1, Wg2, bg2,
           W_out, b_out):
    del edge_type
    src = edge_index[0].astype(jnp.int32)
    dst = edge_index[1].astype(jnp.int32)
    zeros_d = jnp.zeros((RPT, D), jnp.float32)
    ones_w = jnp.ones((CHUNK, D), jnp.float32)

    degp = _deg_kernel(dst, ones_w, zeros_d)

    y0, dinv = pl.pallas_call(
        _tc_in_body,
        out_shape=[
            jax.ShapeDtypeStruct((N, D), jnp.float32),
            jax.ShapeDtypeStruct((N, 1), jnp.float32),
        ],
    )(feature, W_in, b_in, degp)

    zp1 = _prop_kernel(y0, src, dst, zeros_d)

    y1 = pl.pallas_call(
        _tc_mid_body,
        out_shape=jax.ShapeDtypeStruct((N, D), jnp.float32),
    )(zp1, y0, dinv, Wg1, bg1)

    zp2 = _prop_kernel(y1, src, dst, zeros_d)

    out = pl.pallas_call(
        _tc_final_body,
        out_shape=jax.ShapeDtypeStruct((N, OUT), jnp.float32),
    )(zp2, y1, dinv, Wg2, bg2, W_out, b_out)
    return out
